# baseline (speedup 1.0000x reference)
import functools

import jax
import jax.numpy as jnp
from jax import lax
from jax.experimental import pallas as pl
from jax.experimental.pallas import tpu as pltpu
from jax.experimental.pallas import tpu_sc as plsc

B = 16384
ENC = 50
EMB = 50
NOUT = 3
NC = 2
NS = 16
NW = NC * NS
ROWS_PER_W = B // NW
CHUNK = 8
NCHUNK = ROWS_PER_W // CHUNK
LOOKUPS = CHUNK * ENC
UNITS = LOOKUPS * 4
VOCABW = 1000000 * EMB
IDX_PER_GATHER = 128
GATHERS = UNITS // IDX_PER_GATHER
REM = UNITS - GATHERS * IDX_PER_GATHER


def _sc_body(uidx, offs, table16, w_arr, b16, out,
             uidx_v, rows_v, w_v, b_v, out_v, off_v, sem_a, sem_b):
    wid = lax.axis_index("s") * NC + lax.axis_index("c")
    base_row = wid * ROWS_PER_W

    pltpu.sync_copy(w_arr, w_v)
    pltpu.sync_copy(b16, b_v)

    sems = (sem_a, sem_b)

    def stage(c, p):
        base = (wid * NCHUNK + c)
        pltpu.sync_copy(uidx.at[pl.ds(base * UNITS, UNITS)], uidx_v.at[p])
        pltpu.sync_copy(offs.at[pl.ds(base * ENC * 32, ENC * 32)], off_v.at[p])

    def fire(c, p):
        for g in range(GATHERS + 1):
            n = IDX_PER_GATHER if g < GATHERS else REM
            pltpu.async_copy(
                table16.at[uidx_v.at[p, pl.ds(g * IDX_PER_GATHER, n)]],
                rows_v.at[p, pl.ds(g * IDX_PER_GATHER, n), :],
                sems[p])

    def drain(c, p):
        for g in range(GATHERS + 1):
            n = IDX_PER_GATHER if g < GATHERS else REM
            pltpu.make_async_copy(
                table16.at[uidx_v.at[p, pl.ds(g * IDX_PER_GATHER, n)]],
                rows_v.at[p, pl.ds(g * IDX_PER_GATHER, n), :],
                sems[p]).wait()

    def compute(c, p):
        zero = jnp.zeros((16,), jnp.float32)
        accs = (zero,) * (CHUNK * NOUT)

        def jbody(j, accs):
            new = list(accs)
            ov1 = off_v[p, pl.ds(j * 32, 16)]
            ov2 = off_v[p, pl.ds(j * 32 + 16, 16)]
            o_s = [ov1[r] for r in range(CHUNK)]
            c3_s = [ov1[r + CHUNK] for r in range(CHUNK)]
            u3_s = [ov2[r] for r in range(CHUNK)]
            for v in range(4):
                wv = [w_v[j * 12 + v * NOUT + o, :] for o in range(NOUT)]
                for r in range(CHUNK):
                    k = r * ENC + j
                    if v < 3:
                        rv = rows_v[p, k * 4 + v, pl.ds(o_s[r], 16)]
                    else:
                        rv = rows_v[p, k * 4 + u3_s[r], pl.ds(c3_s[r], 16)]
                    for o in range(NOUT):
                        a = r * NOUT + o
                        new[a] = new[a] + rv * wv[o]
            return tuple(new)

        accs = lax.fori_loop(0, ENC, jbody, accs)
        lane = lax.iota(jnp.int32, 16)
        bvec = b_v[:]
        for r in range(CHUNK):
            row = c * CHUNK + r
            out16 = bvec
            for o in range(NOUT):
                s = jnp.sum(accs[r * NOUT + o])
                out16 = out16 + s * (lane == o).astype(jnp.float32)
            out_v[row, :] = out16

    stage(0, 0)
    fire(0, 0)
    stage(1, 1)

    def outer(i, carry):
        for p in range(2):
            c = 2 * i + p
            drain(c, p)

            @pl.when(c + 1 < NCHUNK)
            def _fire_next():
                fire(c + 1, 1 - p)

            compute(c, p)

            @pl.when(c + 2 < NCHUNK)
            def _stage_next():
                stage(c + 2, p)
        return carry

    lax.fori_loop(0, NCHUNK // 2, outer, 0)
    pltpu.sync_copy(out_v, out.at[pl.ds(base_row, ROWS_PER_W), :])


@jax.jit
def kernel(x, emb_table, W, b):
    x_flat = x.reshape(-1).astype(jnp.int32)
    word = x_flat * ENC
    ubase = word >> 4
    uidx = (ubase[:, None] + jnp.arange(4, dtype=jnp.int32)).reshape(-1)
    o = word & 15
    c3 = (o + 34) & 15
    u3 = (o + 34) >> 4

    def _rearr(a):
        return a.reshape(NW * NCHUNK, CHUNK, ENC).transpose(0, 2, 1)

    offs = jnp.concatenate(
        [_rearr(o), _rearr(c3), _rearr(u3),
         jnp.zeros((NW * NCHUNK, ENC, CHUNK), jnp.int32)], axis=-1).reshape(-1)
    one = (b[0] < jnp.float32(jnp.inf)).astype(jnp.float32)
    table16 = (emb_table.reshape(-1) * one).reshape(-1, 16)

    w3 = W.reshape(NOUT, ENC, EMB).transpose(1, 0, 2)
    v012 = w3[:, :, :48].reshape(ENC, NOUT, 3, 16).transpose(0, 2, 1, 3)
    tail = jnp.concatenate(
        [jnp.zeros((ENC, NOUT, 14), jnp.float32), w3[:, :, 48:]], axis=-1)
    w_arr = jnp.concatenate([v012, tail[:, None]], axis=1)
    w_arr = w_arr.reshape(ENC * 4 * NOUT, 16)
    b16 = jnp.concatenate([b, jnp.zeros((16 - NOUT,), jnp.float32)])

    mesh = plsc.VectorSubcoreMesh(core_axis_name="c", subcore_axis_name="s")
    run = pl.kernel(
        _sc_body,
        out_type=jax.ShapeDtypeStruct((B, 16), jnp.float32),
        mesh=mesh,
        compiler_params=pltpu.CompilerParams(
            needs_layout_passes=False, use_tc_tiling_on_sc=False,
            disable_bounds_checks=True),
        scratch_types=[
            pltpu.VMEM((2, UNITS), jnp.int32),
            pltpu.VMEM((2, UNITS, 16), jnp.float32),
            pltpu.VMEM((ENC * 4 * NOUT, 16), jnp.float32),
            pltpu.VMEM((16,), jnp.float32),
            pltpu.VMEM((ROWS_PER_W, 16), jnp.float32),
            pltpu.VMEM((2, ENC * 32), jnp.int32),
            pltpu.SemaphoreType.DMA,
            pltpu.SemaphoreType.DMA,
        ],
    )
    return run(uidx, offs, table16, w_arr, b16)[:, :NOUT]


# speedup vs baseline: 1.2445x; 1.2445x over previous
import functools

import jax
import jax.numpy as jnp
from jax import lax
from jax.experimental import pallas as pl
from jax.experimental.pallas import tpu as pltpu
from jax.experimental.pallas import tpu_sc as plsc

B = 16384
ENC = 50
EMB = 50
NOUT = 3
NC = 2
NS = 16
NW = NC * NS
ROWS_PER_W = B // NW
CHUNK = 8
NCHUNK = ROWS_PER_W // CHUNK
LOOKUPS = CHUNK * ENC
UNITS = LOOKUPS * 4
VOCABW = 1000000 * EMB
IDX_PER_GATHER = 128
GATHERS = UNITS // IDX_PER_GATHER
REM = UNITS - GATHERS * IDX_PER_GATHER


def _sc_body(uidx, table16, w_arr, b16, out,
             uidx_v, rows_v, w_v, b_v, out_v, sem_a, sem_b):
    wid = lax.axis_index("s") * NC + lax.axis_index("c")
    base_row = wid * ROWS_PER_W

    pltpu.sync_copy(w_arr, w_v)
    pltpu.sync_copy(b16, b_v)

    sems = (sem_a, sem_b)

    def stage(c, p):
        base = (wid * NCHUNK + c)
        pltpu.sync_copy(uidx.at[pl.ds(base * UNITS, UNITS)], uidx_v.at[p])

    def fire(c, p):
        for g in range(GATHERS + 1):
            n = IDX_PER_GATHER if g < GATHERS else REM
            pltpu.async_copy(
                table16.at[uidx_v.at[p, pl.ds(g * IDX_PER_GATHER, n)]],
                rows_v.at[p, pl.ds(g * IDX_PER_GATHER, n), :],
                sems[p])

    def drain(c, p):
        for g in range(GATHERS + 1):
            n = IDX_PER_GATHER if g < GATHERS else REM
            pltpu.make_async_copy(
                table16.at[uidx_v.at[p, pl.ds(g * IDX_PER_GATHER, n)]],
                rows_v.at[p, pl.ds(g * IDX_PER_GATHER, n), :],
                sems[p]).wait()

    def compute(c, p):
        zero = jnp.zeros((16,), jnp.float32)
        accs = (zero,) * (CHUNK * NOUT)

        def jbody(j, accs):
            new = list(accs)
            two = j * 0 + 2
            for v in range(4):
                wv = [w_v[j * 12 + v * NOUT + o, :] for o in range(NOUT)]
                for r in range(CHUNK):
                    k = r * ENC + j
                    if v < 3:
                        rv = rows_v[p, k * 4 + v, :]
                    else:
                        rv = rows_v[p, k * 4 + 2, pl.ds(two, 16)]
                    for o in range(NOUT):
                        a = r * NOUT + o
                        new[a] = new[a] + rv * wv[o]
            return tuple(new)

        accs = lax.fori_loop(0, ENC, jbody, accs)
        lane = lax.iota(jnp.int32, 16)
        bvec = b_v[:]
        for r in range(CHUNK):
            row = c * CHUNK + r
            out16 = bvec
            for o in range(NOUT):
                s = jnp.sum(accs[r * NOUT + o])
                out16 = out16 + s * (lane == o).astype(jnp.float32)
            out_v[row, :] = out16

    stage(0, 0)
    fire(0, 0)
    stage(1, 1)

    def outer(i, carry):
        for p in range(2):
            c = 2 * i + p
            drain(c, p)

            @pl.when(c + 1 < NCHUNK)
            def _fire_next():
                fire(c + 1, 1 - p)

            compute(c, p)

            @pl.when(c + 2 < NCHUNK)
            def _stage_next():
                stage(c + 2, p)
        return carry

    lax.fori_loop(0, NCHUNK // 2, outer, 0)
    pltpu.sync_copy(out_v, out.at[pl.ds(base_row, ROWS_PER_W), :])


@jax.jit
def kernel(x, emb_table, W, b):
    x_flat = x.reshape(-1).astype(jnp.int32)
    table_pad = jnp.pad(emb_table, ((0, 0), (0, 128 - EMB)))
    table16 = table_pad.reshape(-1, 16)
    uidx = (x_flat[:, None] * 8
            + jnp.arange(4, dtype=jnp.int32)).reshape(-1)

    w3 = W.reshape(NOUT, ENC, EMB).transpose(1, 0, 2)
    v012 = w3[:, :, :48].reshape(ENC, NOUT, 3, 16).transpose(0, 2, 1, 3)
    tail = jnp.concatenate(
        [jnp.zeros((ENC, NOUT, 14), jnp.float32), w3[:, :, 48:]], axis=-1)
    w_arr = jnp.concatenate([v012, tail[:, None]], axis=1)
    w_arr = w_arr.reshape(ENC * 4 * NOUT, 16)
    b16 = jnp.concatenate([b, jnp.zeros((16 - NOUT,), jnp.float32)])

    mesh = plsc.VectorSubcoreMesh(core_axis_name="c", subcore_axis_name="s")
    run = pl.kernel(
        _sc_body,
        out_type=jax.ShapeDtypeStruct((B, 16), jnp.float32),
        mesh=mesh,
        compiler_params=pltpu.CompilerParams(
            needs_layout_passes=False, use_tc_tiling_on_sc=False,
            disable_bounds_checks=True),
        scratch_types=[
            pltpu.VMEM((2, UNITS), jnp.int32),
            pltpu.VMEM((2, UNITS, 16), jnp.float32),
            pltpu.VMEM((ENC * 4 * NOUT, 16), jnp.float32),
            pltpu.VMEM((16,), jnp.float32),
            pltpu.VMEM((ROWS_PER_W, 16), jnp.float32),
            pltpu.SemaphoreType.DMA,
            pltpu.SemaphoreType.DMA,
        ],
    )
    return run(uidx, table16, w_arr, b16)[:, :NOUT]
